# baseline (speedup 1.0000x reference)
import functools

import jax
import jax.numpy as jnp
from jax.experimental import pallas as pl
from jax.experimental.pallas import tpu as pltpu

_NUM_EXPERTS = 16
_TOP_K = 2
_BLK = 512


def _gate_body(x, w):
    logits = jax.lax.dot_general(
        x, w, (((1,), (1,)), ((), ())), preferred_element_type=jnp.float32
    )
    m = jnp.max(logits, axis=1, keepdims=True)
    e = jnp.exp(logits - m)
    scores = e / jnp.sum(e, axis=1, keepdims=True)

    col = jax.lax.broadcasted_iota(jnp.int32, scores.shape, 1)
    v1 = jnp.max(scores, axis=1, keepdims=True)
    i1 = jnp.min(jnp.where(scores == v1, col, _NUM_EXPERTS), axis=1, keepdims=True)
    masked = jnp.where(col == i1, -jnp.inf, scores)
    v2 = jnp.max(masked, axis=1, keepdims=True)
    i2 = jnp.min(jnp.where(masked == v2, col, _NUM_EXPERTS), axis=1, keepdims=True)

    t = jnp.exp(v2 - v1)
    denom = 1.0 + t
    return jnp.concatenate([1.0 / denom, t / denom], axis=1), jnp.concatenate([i1, i2], axis=1)


def _gate_kernel(xa_ref, xb_ref, w_ref, s_ref, i_ref):
    w = w_ref[...]
    sa, ia = _gate_body(xa_ref[...], w)
    s_ref[: _BLK, :] = sa
    i_ref[: _BLK, :] = ia
    sb, ib = _gate_body(xb_ref[...], w)
    s_ref[_BLK :, :] = sb
    i_ref[_BLK :, :] = ib


@functools.partial(jax.jit, static_argnums=())
def kernel(x, W_gate):
    b, s, h = x.shape
    rows = b * s
    x_flat = x.reshape(rows, h)
    grid = (rows // (2 * _BLK),)
    scores, idx = pl.pallas_call(
        _gate_kernel,
        grid=grid,
        in_specs=[
            pl.BlockSpec((_BLK, h), lambda i: (2 * i, 0)),
            pl.BlockSpec((_BLK, h), lambda i: (2 * i + 1, 0)),
            pl.BlockSpec((_NUM_EXPERTS, h), lambda i: (0, 0)),
        ],
        out_specs=[
            pl.BlockSpec((2 * _BLK, _TOP_K), lambda i: (i, 0)),
            pl.BlockSpec((2 * _BLK, _TOP_K), lambda i: (i, 0)),
        ],
        out_shape=[
            jax.ShapeDtypeStruct((rows, _TOP_K), jnp.float32),
            jax.ShapeDtypeStruct((rows, _TOP_K), jnp.int32),
        ],
        compiler_params=pltpu.CompilerParams(
            dimension_semantics=("arbitrary",),
        ),
    )(x_flat, x_flat, W_gate)
    aux_loss = jnp.array(0.0, dtype=jnp.float32)
    return (scores, idx, aux_loss)


# speedup vs baseline: 1.0849x; 1.0849x over previous
import functools

import jax
import jax.numpy as jnp
from jax.experimental import pallas as pl
from jax.experimental.pallas import tpu as pltpu

_NUM_EXPERTS = 16
_TOP_K = 2
_BLK = 1024


def _gate_kernel(x_ref, w_ref, s_ref, i_ref):
    lt = jax.lax.dot_general(
        w_ref[...], x_ref[...], (((1,), (1,)), ((), ())),
        preferred_element_type=jnp.float32,
    )
    m = jnp.max(lt, axis=0, keepdims=True)
    row = jax.lax.broadcasted_iota(jnp.int32, lt.shape, 0)
    i1 = jnp.min(jnp.where(lt == m, row, _NUM_EXPERTS), axis=0, keepdims=True)
    masked = jnp.where(row == i1, -jnp.inf, lt)
    l2 = jnp.max(masked, axis=0, keepdims=True)
    i2 = jnp.min(jnp.where(masked == l2, row, _NUM_EXPERTS), axis=0, keepdims=True)
    z = jnp.sum(jnp.exp(lt - m), axis=0, keepdims=True)

    v1 = 1.0 / z
    t = jnp.exp(jnp.exp(l2 - m) / z - v1)
    d = 1.0 + t
    s_ref[...] = jnp.concatenate([1.0 / d, t / d], axis=0).T
    i_ref[...] = jnp.concatenate([i1, i2], axis=0).T


@functools.partial(jax.jit, static_argnums=())
def kernel(x, W_gate):
    b, s, h = x.shape
    rows = b * s
    x_flat = x.reshape(rows, h)
    grid = (rows // _BLK,)
    scores, idx = pl.pallas_call(
        _gate_kernel,
        grid=grid,
        in_specs=[
            pl.BlockSpec((_BLK, h), lambda i: (i, 0)),
            pl.BlockSpec((_NUM_EXPERTS, h), lambda i: (0, 0)),
        ],
        out_specs=[
            pl.BlockSpec((_BLK, _TOP_K), lambda i: (i, 0)),
            pl.BlockSpec((_BLK, _TOP_K), lambda i: (i, 0)),
        ],
        out_shape=[
            jax.ShapeDtypeStruct((rows, _TOP_K), jnp.float32),
            jax.ShapeDtypeStruct((rows, _TOP_K), jnp.int32),
        ],
        compiler_params=pltpu.CompilerParams(
            dimension_semantics=("arbitrary",),
        ),
    )(x_flat, W_gate)
    aux_loss = jnp.array(0.0, dtype=jnp.float32)
    return (scores, idx, aux_loss)


# speedup vs baseline: 1.0872x; 1.0021x over previous
import functools

import jax
import jax.numpy as jnp
from jax.experimental import pallas as pl
from jax.experimental.pallas import tpu as pltpu

_NUM_EXPERTS = 16
_TOP_K = 2
_BLK = 1024


def _gate_kernel(x_ref, w_ref, s_ref, i_ref):
    lt = jax.lax.dot_general(
        w_ref[...], x_ref[...], (((1,), (1,)), ((), ())),
        preferred_element_type=jnp.float32,
    )
    m = jnp.max(lt, axis=0, keepdims=True)
    row = jax.lax.broadcasted_iota(jnp.int32, lt.shape, 0)
    i1 = jnp.min(jnp.where(lt == m, row, _NUM_EXPERTS), axis=0, keepdims=True)
    masked = jnp.where(row == i1, -jnp.inf, lt)
    l2 = jnp.max(masked, axis=0, keepdims=True)
    i2 = jnp.min(jnp.where(masked == l2, row, _NUM_EXPERTS), axis=0, keepdims=True)
    z = jnp.sum(jnp.exp(lt - m), axis=0, keepdims=True)

    v1 = 1.0 / z
    t = jnp.exp(jnp.exp(l2 - m) / z - v1)
    d = 1.0 + t
    s_ref[...] = jnp.concatenate([1.0 / d, t / d], axis=0).T
    i_ref[...] = jnp.concatenate([i1, i2], axis=0).T


@functools.partial(jax.jit, static_argnums=())
def kernel(x, W_gate):
    b, s, h = x.shape
    rows = b * s
    x_flat = x.reshape(rows, h)
    grid = (rows // _BLK,)
    scores, idx = pl.pallas_call(
        _gate_kernel,
        grid=grid,
        in_specs=[
            pl.BlockSpec((_BLK, h), lambda i: (i, 0)),
            pl.BlockSpec((_NUM_EXPERTS, h), lambda i: (0, 0)),
        ],
        out_specs=[
            pl.BlockSpec((_BLK, _TOP_K), lambda i: (i, 0)),
            pl.BlockSpec((_BLK, _TOP_K), lambda i: (i, 0)),
        ],
        out_shape=[
            jax.ShapeDtypeStruct((rows, _TOP_K), jnp.float32),
            jax.ShapeDtypeStruct((rows, _TOP_K), jnp.int32),
        ],
        compiler_params=pltpu.CompilerParams(
            dimension_semantics=("parallel",),
        ),
    )(x_flat, W_gate)
    aux_loss = jnp.array(0.0, dtype=jnp.float32)
    return (scores, idx, aux_loss)
